# baseline (speedup 1.0000x reference)
import functools

import jax
import jax.numpy as jnp
from jax import lax
from jax.experimental import pallas as pl
from jax.experimental.pallas import tpu as pltpu
from jax.experimental.pallas import tpu_sc as plsc

_VOCAB = 100000
_D = 128
_L = 2048
_B = 32
_NC = 2
_NS = 16
_NW = _NC * _NS
_LW = _L // _NW
_LANES = 16


def _emb_body(seq_hbm, table_hbm, pos_hbm, out_hbm, idx_v, pos_v, tok_v, gsem,
              isem):
    wid = lax.axis_index("s") * _NC + lax.axis_index("c")
    l0 = wid * _LW

    idx_copies = [
        pltpu.async_copy(seq_hbm.at[pl.ds(b * _L + l0, _LW)], idx_v.at[b], isem)
        for b in range(_B)
    ]
    pltpu.sync_copy(pos_hbm.at[pl.ds(l0 * _D, _LW * _D)], pos_v)
    for c in idx_copies:
        c.wait()

    def batch_body(b, carry):
        pltpu.async_copy(table_hbm.at[idx_v.at[b]], tok_v, gsem).wait()

        def add_row(r, c2):
            for c in range(_D // _LANES):
                plsc.addupdate(
                    tok_v.at[r, pl.ds(c * _LANES, _LANES)],
                    pos_v[pl.ds(r * _D + c * _LANES, _LANES)],
                )
            return c2

        lax.fori_loop(0, _LW, add_row, 0)
        pltpu.sync_copy(tok_v, out_hbm.at[pl.ds(b * _L + l0, _LW)])
        return carry

    lax.fori_loop(0, _B, batch_body, 0)


@jax.jit
def kernel(sequence, token_table, pos_table):
    seq = sequence.astype(jnp.int32)
    pos_flat = pos_table.reshape(_L * _D)
    mesh = plsc.VectorSubcoreMesh(core_axis_name="c", subcore_axis_name="s")
    out = pl.kernel(
        _emb_body,
        out_type=jax.ShapeDtypeStruct((_B * _L, _D), jnp.float32),
        mesh=mesh,
        scratch_types=[
            pltpu.VMEM((_B, _LW), jnp.int32),
            pltpu.VMEM((_LW * _D,), jnp.float32),
            pltpu.VMEM((_LW, _D), jnp.float32),
            pltpu.SemaphoreType.DMA,
            pltpu.SemaphoreType.DMA,
        ],
    )(seq.reshape(_B * _L), token_table, pos_flat)
    return out.reshape(_B, _L, _D)


# speedup vs baseline: 1.7617x; 1.7617x over previous
import jax
import jax.numpy as jnp
from jax import lax
from jax.experimental import pallas as pl
from jax.experimental.pallas import tpu as pltpu
from jax.experimental.pallas import tpu_sc as plsc

_VOCAB = 100000
_D = 128
_L = 2048
_B = 32
_NC = 2
_NS = 16
_NW = _NC * _NS
_LW = _L // _NW
_LANES = 16
_NBUF = 4
_GROUPS = _B // _NBUF


def _emb_body(seq_hbm, table_hbm, pos_hbm, out_hbm, idx_v, pos_v, tok_v,
              isem, gsems, osems):
    wid = lax.axis_index("s") * _NC + lax.axis_index("c")
    l0 = wid * _LW

    idx_copies = [
        pltpu.async_copy(seq_hbm.at[pl.ds(b * _L + l0, _LW)], idx_v.at[b], isem)
        for b in range(_B)
    ]
    pltpu.sync_copy(pos_hbm.at[pl.ds(l0 * _D, _LW * _D)], pos_v)
    for c in idx_copies:
        c.wait()

    def gstart(b, buf):
        pltpu.async_copy(table_hbm.at[idx_v.at[b]], tok_v.at[buf], gsems[buf])

    def gwait(buf):
        pltpu.make_async_copy(
            table_hbm.at[pl.ds(0, _LW)], tok_v.at[buf], gsems[buf]).wait()

    def ostart(b, buf):
        pltpu.async_copy(
            tok_v.at[buf], out_hbm.at[pl.ds(b * _L + l0, _LW)], osems[buf])

    def owait(buf):
        pltpu.make_async_copy(
            tok_v.at[buf], out_hbm.at[pl.ds(0, _LW)], osems[buf]).wait()

    def add_pos(buf):
        @pl.loop(0, _LW, unroll=8)
        def _(r):
            for c in range(_D // _LANES):
                plsc.addupdate(
                    tok_v.at[buf, r, pl.ds(c * _LANES, _LANES)],
                    pos_v[pl.ds(r * _D + c * _LANES, _LANES)],
                )

    for b in range(_NBUF):
        gstart(b, b)

    def group(i, carry):
        for j in range(_NBUF):
            b = _NBUF * i + j
            s = b - 2
            sbuf = (j - 2) % _NBUF

            @pl.when(jnp.logical_and(s >= 0, s + _NBUF < _B))
            def _():
                owait(sbuf)
                gstart(s + _NBUF, sbuf)

            gwait(j)
            add_pos(j)
            ostart(b, j)
        return carry

    lax.fori_loop(0, _GROUPS, group, 0)
    for j in range(_NBUF):
        owait(j)


@jax.jit
def kernel(sequence, token_table, pos_table):
    seq = sequence.astype(jnp.int32)
    pos_flat = pos_table.reshape(_L * _D)
    mesh = plsc.VectorSubcoreMesh(core_axis_name="c", subcore_axis_name="s")
    out = pl.kernel(
        _emb_body,
        out_type=jax.ShapeDtypeStruct((_B * _L, _D), jnp.float32),
        mesh=mesh,
        scratch_types=[
            pltpu.VMEM((_B, _LW), jnp.int32),
            pltpu.VMEM((_LW * _D,), jnp.float32),
            pltpu.VMEM((_NBUF, _LW, _D), jnp.float32),
            pltpu.SemaphoreType.DMA,
            [pltpu.SemaphoreType.DMA] * _NBUF,
            [pltpu.SemaphoreType.DMA] * _NBUF,
        ],
    )(seq.reshape(_B * _L), token_table, pos_flat)
    return out.reshape(_B, _L, _D)
